# Initial kernel scaffold; baseline (speedup 1.0000x reference)
import jax
import jax.numpy as jnp
from jax.experimental import pallas as pl


def kernel(decoder_input, encoder_out, hyps_log_prob, emb, conv_w, Wp, bp, Wj, bj):
    raise NotImplementedError("write your pallas kernel here")


# speedup vs baseline: 25.9751x; 25.9751x over previous
import functools

import jax
import jax.numpy as jnp
from jax import lax
from jax.experimental import pallas as pl
from jax.experimental.pallas import tpu as pltpu

_V = 100000
_D = 512
_N = 64
_CTX = 2
_G = _D // 4
_BEAM = 8
_BV = 4000
_NB = _V // _BV
_NEG = -1e30
_IBIG = 2 ** 30


def _body(e01_ref, enc_ref, hlp_ref, m0_ref, m1_ref, wp_ref, bp_ref,
          wj_ref, bj_ref,
          outv_ref, outp_ref, outh_ref, outt_ref,
          joint_s, m_s, s_s, candv_s, candi_s):
    i = pl.program_id(0)

    @pl.when(i == 0)
    def _init():
        e0 = e01_ref[0:_N, :]
        e1 = e01_ref[_N:2 * _N, :]
        nn = (((1,), (0,)), ((), ()))
        d = lax.dot_general(e0, m0_ref[...], nn,
                            preferred_element_type=jnp.float32)
        d += lax.dot_general(e1, m1_ref[...], nn,
                             preferred_element_type=jnp.float32)
        d = jnp.maximum(d, 0.0)
        nt = (((1,), (1,)), ((), ()))
        p = lax.dot_general(d, wp_ref[...], nt,
                            preferred_element_type=jnp.float32)
        joint_s[...] = jnp.tanh(enc_ref[...] + p + bp_ref[...])
        m_s[...] = jnp.full((_N, 128), _NEG, jnp.float32)
        s_s[...] = jnp.zeros((_N, 128), jnp.float32)
        candv_s[...] = jnp.full((_N, 128), _NEG, jnp.float32)
        candi_s[...] = jnp.zeros((_N, 128), jnp.int32)

    nt = (((1,), (1,)), ((), ()))
    logits = lax.dot_general(joint_s[...], wj_ref[...], nt,
                             preferred_element_type=jnp.float32)
    logits += lax.dot_general(jnp.ones((_N, 1), jnp.float32), bj_ref[0], nt,
                              preferred_element_type=jnp.float32)

    bm = jnp.max(logits, axis=1, keepdims=True)
    m_old = m_s[...][:, 0:1]
    m_new = jnp.maximum(m_old, bm)
    sumexp = jnp.sum(jnp.exp(logits - m_new), axis=1, keepdims=True)
    s_new = s_s[...][:, 0:1] * jnp.exp(m_old - m_new) + sumexp
    m_s[...] = jnp.broadcast_to(m_new, (_N, 128))
    s_s[...] = jnp.broadcast_to(s_new, (_N, 128))

    lane = lax.broadcasted_iota(jnp.int32, (_N, 128), 1)
    colid = lax.broadcasted_iota(jnp.int32, (_N, _BV), 1)
    candv = candv_s[...]
    candi = candi_s[...]
    v = logits
    for k in range(_BEAM):
        mk = jnp.max(v, axis=1, keepdims=True)
        c = jnp.min(jnp.where(v == mk, colid, _IBIG), axis=1, keepdims=True)
        candv = jnp.where(lane == _BEAM + k, mk, candv)
        candi = jnp.where(lane == _BEAM + k, i * _BV + c, candi)
        v = jnp.where(colid == c, _NEG, v)

    w = jnp.where(lane < 16, candv, _NEG)
    nv = jnp.full((_N, 128), _NEG, jnp.float32)
    ni = jnp.zeros((_N, 128), jnp.int32)
    for k in range(_BEAM):
        mk = jnp.max(w, axis=1, keepdims=True)
        c = jnp.min(jnp.where(w == mk, lane, _IBIG), axis=1, keepdims=True)
        ci = jnp.min(jnp.where(lane == c, candi, _IBIG), axis=1, keepdims=True)
        nv = jnp.where(lane == k, mk, nv)
        ni = jnp.where(lane == k, ci, ni)
        w = jnp.where(lane == c, _NEG, w)
    candv_s[...] = nv
    candi_s[...] = ni

    @pl.when(i == _NB - 1)
    def _final():
        lane2 = lax.broadcasted_iota(jnp.int32, (_N, 128), 1)
        rowi = lax.broadcasted_iota(jnp.int32, (_N, 128), 0)
        m = m_s[...][:, 0:1]
        s = s_s[...][:, 0:1]
        lse = m + jnp.log(s)
        hlp_b = hlp_ref[...]
        adj = jnp.where(lane2 < _BEAM,
                        candv_s[...] + hlp_b - lse, _NEG)
        ci = candi_s[...]
        flat = rowi * _V + ci
        laneo = lax.broadcasted_iota(jnp.int32, (1, 128), 1)
        ov = jnp.full((1, 128), 0.0, jnp.float32)
        op = jnp.full((1, 128), 0.0, jnp.float32)
        oh = jnp.zeros((1, 128), jnp.int32)
        ot = jnp.zeros((1, 128), jnp.int32)
        for k in range(_BEAM):
            g = jnp.max(adj)
            hit = adj == g
            f = jnp.min(jnp.where(hit, flat, _IBIG))
            sel = hit & (flat == f)
            hlp_at = jnp.min(jnp.where(sel, hlp_b, jnp.float32(_IBIG)))
            tok = jnp.min(jnp.where(sel, ci, _IBIG))
            hyp = jnp.min(jnp.where(sel, rowi, _IBIG))
            ov = jnp.where(laneo == k, g, ov)
            op = jnp.where(laneo == k, jnp.exp(g - hlp_at), op)
            oh = jnp.where(laneo == k, hyp, oh)
            ot = jnp.where(laneo == k, tok, ot)
            adj = jnp.where(sel, _NEG, adj)
        outv_ref[...] = ov
        outp_ref[...] = op
        outh_ref[...] = oh
        outt_ref[...] = ot


def kernel(decoder_input, encoder_out, hyps_log_prob, emb, conv_w, Wp, bp,
           Wj, bj):
    f32 = jnp.float32
    cw = conv_w.reshape(_G, 4, 4, _CTX)
    eye = jnp.eye(_G, dtype=f32)
    m0 = jnp.einsum('goi,gh->giho', cw[..., 0], eye).reshape(_D, _D)
    m1 = jnp.einsum('goi,gh->giho', cw[..., 1], eye).reshape(_D, _D)

    ids = decoder_input.T.reshape(-1)
    e01 = jnp.take(emb, ids, axis=0)

    hlp_b = jnp.broadcast_to(hyps_log_prob, (_N, 128))
    bp2 = bp.reshape(1, _D)
    bj2 = bj.reshape(_NB, _BV, 1)

    outs = pl.pallas_call(
        _body,
        grid=(_NB,),
        in_specs=[
            pl.BlockSpec((2 * _N, _D), lambda i: (0, 0)),
            pl.BlockSpec((_N, _D), lambda i: (0, 0)),
            pl.BlockSpec((_N, 128), lambda i: (0, 0)),
            pl.BlockSpec((_D, _D), lambda i: (0, 0)),
            pl.BlockSpec((_D, _D), lambda i: (0, 0)),
            pl.BlockSpec((_D, _D), lambda i: (0, 0)),
            pl.BlockSpec((1, _D), lambda i: (0, 0)),
            pl.BlockSpec((_BV, _D), lambda i: (i, 0)),
            pl.BlockSpec((1, _BV, 1), lambda i: (i, 0, 0)),
        ],
        out_specs=[pl.BlockSpec((1, 128), lambda i: (0, 0))] * 4,
        out_shape=[
            jax.ShapeDtypeStruct((1, 128), f32),
            jax.ShapeDtypeStruct((1, 128), f32),
            jax.ShapeDtypeStruct((1, 128), jnp.int32),
            jax.ShapeDtypeStruct((1, 128), jnp.int32),
        ],
        scratch_shapes=[
            pltpu.VMEM((_N, _D), f32),
            pltpu.VMEM((_N, 128), f32),
            pltpu.VMEM((_N, 128), f32),
            pltpu.VMEM((_N, 128), f32),
            pltpu.VMEM((_N, 128), jnp.int32),
        ],
    )(e01, encoder_out, hlp_b, m0, m1, Wp, bp2, Wj, bj2)

    ov, op, oh, ot = outs
    return (ov[0, :_BEAM], op[0, :_BEAM], oh[0, :_BEAM], ot[0, :_BEAM])
